# Initial kernel scaffold; baseline (speedup 1.0000x reference)
import jax
import jax.numpy as jnp
from jax.experimental import pallas as pl


def kernel(x, edge_index, W, att_src, att_dst, bias):
    raise NotImplementedError("write your pallas kernel here")


# speedup vs baseline: 1.0651x; 1.0651x over previous
import jax
import jax.numpy as jnp
from jax.experimental import pallas as pl

_N = 10000
_E = 160000
_D = 256
_H = 8
_C = _D // _H

_ROW_BLK = 2000


def _mm_body(x_ref, w_ref, att_ref, h_ref, a_ref):
    h = jnp.dot(x_ref[...], w_ref[...], preferred_element_type=jnp.float32)
    h_ref[...] = h
    a_ref[...] = jnp.dot(h, att_ref[...], preferred_element_type=jnp.float32)


def _dense(x, W, att_cols):
    n = x.shape[0]
    grid = (n // _ROW_BLK,)
    return pl.pallas_call(
        _mm_body,
        grid=grid,
        in_specs=[
            pl.BlockSpec((_ROW_BLK, _D), lambda i: (i, 0)),
            pl.BlockSpec((_D, _D), lambda i: (0, 0)),
            pl.BlockSpec((_D, 2 * _H), lambda i: (0, 0)),
        ],
        out_specs=[
            pl.BlockSpec((_ROW_BLK, _D), lambda i: (i, 0)),
            pl.BlockSpec((_ROW_BLK, 2 * _H), lambda i: (i, 0)),
        ],
        out_shape=[
            jax.ShapeDtypeStruct((n, _D), jnp.float32),
            jax.ShapeDtypeStruct((n, 2 * _H), jnp.float32),
        ],
    )(x, W, att_cols)


def kernel(x, edge_index, W, att_src, att_dst, bias):
    n = x.shape[0]
    loops = jnp.arange(n, dtype=edge_index.dtype)
    src = jnp.concatenate([edge_index[0], loops])
    dst = jnp.concatenate([edge_index[1], loops])

    As = jnp.zeros((_H, _C, _H), jnp.float32).at[
        jnp.arange(_H), :, jnp.arange(_H)].set(att_src).reshape(_D, _H)
    Ad = jnp.zeros((_H, _C, _H), jnp.float32).at[
        jnp.arange(_H), :, jnp.arange(_H)].set(att_dst).reshape(_D, _H)
    att_cols = jnp.concatenate([As, Ad], axis=1)

    h = x
    for _ in range(2):
        h, a = _dense(h, W, att_cols)
        a_src = a[:, :_H]
        a_dst = a[:, _H:]
        alpha = a_src[src] + a_dst[dst]
        alpha = jnp.where(alpha >= 0, alpha, 0.2 * alpha)
        ex = jnp.exp(alpha)
        denom = jax.ops.segment_sum(ex, dst, num_segments=n)
        attw = ex / denom[dst]
        msg = h[src].reshape(-1, _H, _C) * attw[:, :, None]
        h = jax.ops.segment_sum(msg, dst, num_segments=n).reshape(n, _D)
    return h + bias


# speedup vs baseline: 31.0569x; 29.1595x over previous
import jax
import jax.numpy as jnp
from jax import lax
from jax.experimental import pallas as pl
from jax.experimental.pallas import tpu as pltpu
from jax.experimental.pallas import tpu_sc as plsc

_N = 10000
_E = 160000
_D = 256
_H = 8
_C = _D // _H
_HALF = _D // 2

_EREAL = _E + _N
_K = 128
_NCH = 84
_T = _NCH * _K
_EP = 16 * _T
_PAD = _EP - _EREAL
_NT = 16
_NPB = 624
_NTAIL = _N - _NT * _NPB
_NB = 48

_ROW_BLK = 2000


def _mm_body(xlo_ref, xhi_ref, w_ref, att_ref, h2_ref, a_ref):
    h = jnp.dot(xlo_ref[...], w_ref[:_HALF, :],
                preferred_element_type=jnp.float32)
    h += jnp.dot(xhi_ref[...], w_ref[_HALF:, :],
                 preferred_element_type=jnp.float32)
    h2_ref[0] = h[:, :_HALF]
    h2_ref[1] = h[:, _HALF:]
    a_ref[...] = jnp.dot(h, att_ref[...], preferred_element_type=jnp.float32)


def _dense(xlo, xhi, W, att_cols):
    grid = (_N // _ROW_BLK,)
    return pl.pallas_call(
        _mm_body,
        grid=grid,
        in_specs=[
            pl.BlockSpec((_ROW_BLK, _HALF), lambda i: (i, 0)),
            pl.BlockSpec((_ROW_BLK, _HALF), lambda i: (i, 0)),
            pl.BlockSpec((_D, _D), lambda i: (0, 0)),
            pl.BlockSpec((_D, 2 * _H), lambda i: (0, 0)),
        ],
        out_specs=[
            pl.BlockSpec((2, _ROW_BLK, _HALF), lambda i: (0, i, 0)),
            pl.BlockSpec((_ROW_BLK, 2 * _H), lambda i: (i, 0)),
        ],
        out_shape=[
            jax.ShapeDtypeStruct((2, _N, _HALF), jnp.float32),
            jax.ShapeDtypeStruct((_N, 2 * _H), jnp.float32),
        ],
    )(xlo, xhi, W, att_cols)


def _sc_body(h2, a2, srcp, dstp, z16, z128, out,
             src_ch, dst_loc, abdst, exbuf, rows, denbuf,
             denom_sh, out_sh):
    c = lax.axis_index("c")
    s = lax.axis_index("s")
    lane = lax.iota(jnp.int32, 16)
    l4 = lane >> 2
    lh = lane & 3
    zero16 = lane * 0

    pltpu.sync_copy(dstp.at[s], dst_loc)
    pltpu.sync_copy(z16.at[pl.ds(s * _NPB, _NPB)],
                    denom_sh.at[pl.ds(s * _NPB, _NPB)])
    pltpu.sync_copy(z128.at[pl.ds(s * _NPB, _NPB)],
                    out_sh.at[pl.ds(s * _NPB, _NPB)])

    @pl.when(s == 0)
    def _():
        pltpu.sync_copy(z16.at[pl.ds(_NT * _NPB, _NTAIL)],
                        denom_sh.at[pl.ds(_NT * _NPB, _NTAIL)])
        pltpu.sync_copy(z128.at[pl.ds(_NT * _NPB, _NTAIL)],
                        out_sh.at[pl.ds(_NT * _NPB, _NTAIL)])

    plsc.subcore_barrier()

    acol = 8 * c
    roff = c * _N

    def sweep(ch, carry):
        pltpu.sync_copy(srcp.at[s, ch], src_ch)
        pltpu.sync_copy(a2.at[src_ch], exbuf)
        pltpu.sync_copy(a2.at[dst_loc.at[ch]], abdst)

        def add_roff(j, carry2):
            src_ch[pl.ds(j * 16, 16)] = src_ch[pl.ds(j * 16, 16)] + roff
            return carry2

        lax.fori_loop(0, _K // 16, add_roff, 0, unroll=True)
        pltpu.sync_copy(h2.at[src_ch], rows)

        gbase = (s * _NCH + ch) * _K

        def quad(i, carry2):
            r = i * 4 + l4
            asv = plsc.load_gather(exbuf, [r, acol + lh])
            adv = plsc.load_gather(abdst, [r, acol + 4 + lh])
            al = asv + adv
            al = jnp.where(al >= 0.0, al, al * 0.2)
            exv = jnp.exp(al)
            exv = jnp.where(gbase + r < _EREAL, exv, 0.0)
            plsc.store_scatter(exbuf, [r, lh], exv)
            return carry2

        lax.fori_loop(0, _K // 4, quad, 0, unroll=2)
        pltpu.sync_copy(exbuf, denom_sh.at[dst_loc.at[ch]], add=True)

        def edge(e, carry2):
            for hh in range(4):
                w = plsc.load_gather(exbuf, [zero16 + e, zero16 + hh])
                for v in range(2):
                    sl = pl.ds((2 * hh + v) * 16, 16)
                    rows[e, sl] = rows[e, sl] * w
            return carry2

        lax.fori_loop(0, _K, edge, 0, unroll=2)
        pltpu.sync_copy(rows, out_sh.at[dst_loc.at[ch]], add=True)
        return carry

    lax.fori_loop(0, _NCH, sweep, 0)
    plsc.subcore_barrier()

    def norm_block(base, nrows):
        pltpu.sync_copy(out_sh.at[pl.ds(base, nrows)],
                        rows.at[pl.ds(0, nrows)])
        pltpu.sync_copy(denom_sh.at[pl.ds(base, nrows)],
                        denbuf.at[pl.ds(0, nrows)])

        def recip(i, carry2):
            r = (i * 16 + lane) >> 2
            cl = lane & 3
            dv = plsc.load_gather(denbuf, [r, cl])
            plsc.store_scatter(denbuf, [r, cl], 1.0 / dv)
            return carry2

        lax.fori_loop(0, nrows * 4 // 16, recip, 0)

        def row(rr, carry2):
            for hh in range(4):
                rv = plsc.load_gather(denbuf, [zero16 + rr, zero16 + hh])
                for v in range(2):
                    sl = pl.ds((2 * hh + v) * 16, 16)
                    rows[rr, sl] = rows[rr, sl] * rv
            return carry2

        lax.fori_loop(0, nrows, row, 0, unroll=2)
        pltpu.sync_copy(rows.at[pl.ds(0, nrows)],
                        out.at[c, pl.ds(base, nrows)])

    def norm(b, carry):
        norm_block(s * _NPB + b * _NB, _NB)
        return carry

    lax.fori_loop(0, _NPB // _NB, norm, 0)

    @pl.when(s == 0)
    def _():
        norm_block(_NT * _NPB, _NTAIL)


_sc_call = pl.kernel(
    _sc_body,
    out_type=jax.ShapeDtypeStruct((2, _N, _HALF), jnp.float32),
    mesh=plsc.VectorSubcoreMesh(core_axis_name="c", subcore_axis_name="s"),
    compiler_params=pltpu.CompilerParams(
        use_tc_tiling_on_sc=False, needs_layout_passes=False),
    scratch_types=[
        pltpu.VMEM((_K,), jnp.int32),
        pltpu.VMEM((_NCH, _K), jnp.int32),
        pltpu.VMEM((_K, 2 * _H), jnp.float32),
        pltpu.VMEM((_K, 2 * _H), jnp.float32),
        pltpu.VMEM((_K, _HALF), jnp.float32),
        pltpu.VMEM((_NB, 2 * _H), jnp.float32),
        pltpu.VMEM_SHARED((_N, 2 * _H), jnp.float32),
        pltpu.VMEM_SHARED((_N, _HALF), jnp.float32),
    ],
)


def kernel(x, edge_index, W, att_src, att_dst, bias):
    As = jnp.zeros((_H, _C, _H), jnp.float32).at[
        jnp.arange(_H), :, jnp.arange(_H)].set(att_src).reshape(_D, _H)
    Ad = jnp.zeros((_H, _C, _H), jnp.float32).at[
        jnp.arange(_H), :, jnp.arange(_H)].set(att_dst).reshape(_D, _H)
    att_cols = jnp.concatenate(
        [As[:, :4], Ad[:, :4], As[:, 4:], Ad[:, 4:]], axis=1)

    loops = jnp.arange(_N, dtype=jnp.int32)
    padi = jnp.arange(_PAD, dtype=jnp.int32)
    src = jnp.concatenate([edge_index[0], loops, (padi * 37) % _N])
    dst = jnp.concatenate([edge_index[1], loops, (padi * 41) % _N])
    srcp = src.reshape(_NT, _NCH, _K)
    dstp = dst.reshape(_NT, _NCH, _K)

    z16 = jnp.zeros((_N, 2 * _H), jnp.float32)
    z128 = jnp.zeros((_N, _HALF), jnp.float32)

    xlo = x[:, :_HALF]
    xhi = x[:, _HALF:]
    for _ in range(2):
        h2, a2 = _dense(xlo, xhi, W, att_cols)
        out2 = _sc_call(h2.reshape(2 * _N, _HALF), a2, srcp, dstp, z16, z128)
        xlo = out2[0] + bias[:_HALF]
        xhi = out2[1] + bias[_HALF:]
    return jnp.concatenate([xlo, xhi], axis=1)


# speedup vs baseline: 44.4283x; 1.4305x over previous
import jax
import jax.numpy as jnp
from jax import lax
from jax.experimental import pallas as pl
from jax.experimental.pallas import tpu as pltpu
from jax.experimental.pallas import tpu_sc as plsc

_N = 10000
_E = 160000
_D = 256
_H = 8
_C = _D // _H
_HALF = _D // 2

_EREAL = _E + _N
_K = 112
_NCH = 96
_T = _NCH * _K
_EP = 16 * _T
_PAD = _EP - _EREAL
_NT = 16
_NPB = 624
_NTAIL = _N - _NT * _NPB
_NB = 48

_ROW_BLK = 2000


def _mm_body(xlo_ref, xhi_ref, w_ref, att_ref, h2_ref, a_ref):
    h = jnp.dot(xlo_ref[...], w_ref[:_HALF, :],
                preferred_element_type=jnp.float32)
    h += jnp.dot(xhi_ref[...], w_ref[_HALF:, :],
                 preferred_element_type=jnp.float32)
    h2_ref[0] = h[:, :_HALF]
    h2_ref[1] = h[:, _HALF:]
    a_ref[...] = jnp.dot(h, att_ref[...], preferred_element_type=jnp.float32)


def _dense(xlo, xhi, W, att_cols):
    grid = (_N // _ROW_BLK,)
    return pl.pallas_call(
        _mm_body,
        grid=grid,
        in_specs=[
            pl.BlockSpec((_ROW_BLK, _HALF), lambda i: (i, 0)),
            pl.BlockSpec((_ROW_BLK, _HALF), lambda i: (i, 0)),
            pl.BlockSpec((_D, _D), lambda i: (0, 0)),
            pl.BlockSpec((_D, 2 * _H), lambda i: (0, 0)),
        ],
        out_specs=[
            pl.BlockSpec((2, _ROW_BLK, _HALF), lambda i: (0, i, 0)),
            pl.BlockSpec((_ROW_BLK, 2 * _H), lambda i: (i, 0)),
        ],
        out_shape=[
            jax.ShapeDtypeStruct((2, _N, _HALF), jnp.float32),
            jax.ShapeDtypeStruct((_N, 2 * _H), jnp.float32),
        ],
    )(xlo, xhi, W, att_cols)


def _sc_body(h2, a2d, srcp2, dstp, z16, z128, out,
             src_ch, dst_ch, exb0, exb1, abd0, abd1, rows0, rows1, denbuf,
             semi0, semi1, semg0, semg1,
             denom_sh, out_sh):
    c = lax.axis_index("c")
    s = lax.axis_index("s")
    lane = lax.iota(jnp.int32, 16)
    l4 = lane >> 2
    lh = lane & 3
    zero16 = lane * 0

    EXB = (exb0, exb1)
    ABD = (abd0, abd1)
    ROWS = (rows0, rows1)
    SEMI = (semi0, semi1)
    SEMG = (semg0, semg1)

    pltpu.sync_copy(z16.at[pl.ds(s * _NPB, _NPB)],
                    denom_sh.at[pl.ds(s * _NPB, _NPB)])
    pltpu.sync_copy(z128.at[pl.ds(s * _NPB, _NPB)],
                    out_sh.at[pl.ds(s * _NPB, _NPB)])

    @pl.when(s == 0)
    def _():
        pltpu.sync_copy(z16.at[pl.ds(_NT * _NPB, _NTAIL)],
                        denom_sh.at[pl.ds(_NT * _NPB, _NTAIL)])
        pltpu.sync_copy(z128.at[pl.ds(_NT * _NPB, _NTAIL)],
                        out_sh.at[pl.ds(_NT * _NPB, _NTAIL)])

    plsc.subcore_barrier()

    acol = 8 * c

    def issue_idx(ch, b):
        pltpu.async_copy(srcp2.at[c, s, ch], src_ch.at[b], SEMI[b])
        pltpu.async_copy(dstp.at[s, ch], dst_ch.at[b], SEMI[b])

    def wait_idx(b):
        pltpu.make_async_copy(srcp2.at[c, s, 0], src_ch.at[b], SEMI[b]).wait()
        pltpu.make_async_copy(dstp.at[s, 0], dst_ch.at[b], SEMI[b]).wait()

    def issue_dat(b):
        pltpu.async_copy(a2d.at[src_ch.at[b]], EXB[b], SEMG[b])
        pltpu.async_copy(a2d.at[dst_ch.at[b]], ABD[b], SEMG[b])
        pltpu.async_copy(h2.at[src_ch.at[b]], ROWS[b], SEMG[b])

    def wait_dat(b):
        pltpu.make_async_copy(a2d.at[src_ch.at[b]], EXB[b], SEMG[b]).wait()
        pltpu.make_async_copy(a2d.at[dst_ch.at[b]], ABD[b], SEMG[b]).wait()
        pltpu.make_async_copy(h2.at[src_ch.at[b]], ROWS[b], SEMG[b]).wait()

    issue_idx(0, 0)
    wait_idx(0)
    issue_dat(0)
    issue_idx(1, 1)

    def step(ch, b):
        nb = 1 - b
        wait_dat(b)

        @pl.when(ch + 1 < _NCH)
        def _():
            wait_idx(nb)
            issue_dat(nb)

        gbase = (s * _NCH + ch) * _K

        def quad(i, carry2):
            r = i * 4 + l4
            asv = plsc.load_gather(EXB[b], [r, acol + lh])
            adv = plsc.load_gather(ABD[b], [r, acol + 4 + lh])
            al = asv + adv
            al = jnp.where(al >= 0.0, al, al * 0.2)
            exv = jnp.exp(al)
            exv = jnp.where(gbase + r < _EREAL, exv, 0.0)
            plsc.store_scatter(EXB[b], [r, lh], exv)
            return carry2

        lax.fori_loop(0, _K // 4, quad, 0, unroll=2)
        pltpu.sync_copy(EXB[b], denom_sh.at[dst_ch.at[b]], add=True)

        def edge(e, carry2):
            for hh in range(4):
                w = plsc.load_gather(EXB[b], [zero16 + e, zero16 + hh])
                for v in range(2):
                    sl = pl.ds((2 * hh + v) * 16, 16)
                    ROWS[b][e, sl] = ROWS[b][e, sl] * w
            return carry2

        lax.fori_loop(0, _K, edge, 0, unroll=2)
        pltpu.sync_copy(ROWS[b], out_sh.at[dst_ch.at[b]], add=True)

        @pl.when(ch + 2 < _NCH)
        def _():
            issue_idx(ch + 2, b)

    def pair(i, carry):
        step(2 * i, 0)
        step(2 * i + 1, 1)
        return carry

    lax.fori_loop(0, _NCH // 2, pair, 0)
    plsc.subcore_barrier()

    def norm_block(base, nrows):
        pltpu.sync_copy(out_sh.at[pl.ds(base, nrows)],
                        rows0.at[pl.ds(0, nrows)])
        pltpu.sync_copy(denom_sh.at[pl.ds(base, nrows)],
                        denbuf.at[pl.ds(0, nrows)])

        def recip(i, carry2):
            r = (i * 16 + lane) >> 2
            cl = lane & 3
            dv = plsc.load_gather(denbuf, [r, cl])
            plsc.store_scatter(denbuf, [r, cl], 1.0 / dv)
            return carry2

        lax.fori_loop(0, nrows * 4 // 16, recip, 0)

        def row(rr, carry2):
            for hh in range(4):
                rv = plsc.load_gather(denbuf, [zero16 + rr, zero16 + hh])
                for v in range(2):
                    sl = pl.ds((2 * hh + v) * 16, 16)
                    rows0[rr, sl] = rows0[rr, sl] * rv
            return carry2

        lax.fori_loop(0, nrows, row, 0, unroll=2)
        pltpu.sync_copy(rows0.at[pl.ds(0, nrows)],
                        out.at[c, pl.ds(base, nrows)])

    def norm(bb, carry):
        norm_block(s * _NPB + bb * _NB, _NB)
        return carry

    lax.fori_loop(0, _NPB // _NB, norm, 0)

    @pl.when(s == 0)
    def _():
        norm_block(_NT * _NPB, _NTAIL)


_sc_call = pl.kernel(
    _sc_body,
    out_type=jax.ShapeDtypeStruct((2, _N, _HALF), jnp.float32),
    mesh=plsc.VectorSubcoreMesh(core_axis_name="c", subcore_axis_name="s"),
    compiler_params=pltpu.CompilerParams(
        use_tc_tiling_on_sc=False, needs_layout_passes=False),
    scratch_types=[
        pltpu.VMEM((2, _K), jnp.int32),
        pltpu.VMEM((2, _K), jnp.int32),
        pltpu.VMEM((_K, 2 * _H), jnp.float32),
        pltpu.VMEM((_K, 2 * _H), jnp.float32),
        pltpu.VMEM((_K, 2 * _H), jnp.float32),
        pltpu.VMEM((_K, 2 * _H), jnp.float32),
        pltpu.VMEM((_K, _HALF), jnp.float32),
        pltpu.VMEM((_K, _HALF), jnp.float32),
        pltpu.VMEM((_NB, 2 * _H), jnp.float32),
        pltpu.SemaphoreType.DMA,
        pltpu.SemaphoreType.DMA,
        pltpu.SemaphoreType.DMA,
        pltpu.SemaphoreType.DMA,
        pltpu.VMEM_SHARED((_N, 2 * _H), jnp.float32),
        pltpu.VMEM_SHARED((_N, _HALF), jnp.float32),
    ],
)


def kernel(x, edge_index, W, att_src, att_dst, bias):
    As = jnp.zeros((_H, _C, _H), jnp.float32).at[
        jnp.arange(_H), :, jnp.arange(_H)].set(att_src).reshape(_D, _H)
    Ad = jnp.zeros((_H, _C, _H), jnp.float32).at[
        jnp.arange(_H), :, jnp.arange(_H)].set(att_dst).reshape(_D, _H)
    att_cols = jnp.concatenate(
        [As[:, :4], Ad[:, :4], As[:, 4:], Ad[:, 4:]], axis=1)

    loops = jnp.arange(_N, dtype=jnp.int32)
    padi = jnp.arange(_PAD, dtype=jnp.int32)
    src = jnp.concatenate([edge_index[0], loops, (padi * 37) % _N])
    dst = jnp.concatenate([edge_index[1], loops, (padi * 41) % _N])
    srcp = src.reshape(_NT, _NCH, _K)
    srcp2 = jnp.stack([srcp, srcp + _N])
    dstp = dst.reshape(_NT, _NCH, _K)

    z16 = jnp.zeros((_N, 2 * _H), jnp.float32)
    z128 = jnp.zeros((_N, _HALF), jnp.float32)

    xlo = x[:, :_HALF]
    xhi = x[:, _HALF:]
    for _ in range(2):
        h2, a2 = _dense(xlo, xhi, W, att_cols)
        a2d = jnp.concatenate([a2, a2], axis=0)
        out2 = _sc_call(h2.reshape(2 * _N, _HALF), a2d, srcp2, dstp,
                        z16, z128)
        xlo = out2[0] + bias[:_HALF]
        xhi = out2[1] + bias[_HALF:]
    return jnp.concatenate([xlo, xhi], axis=1)


# speedup vs baseline: 50.1583x; 1.1290x over previous
import jax
import jax.numpy as jnp
from jax import lax
from jax.experimental import pallas as pl
from jax.experimental.pallas import tpu as pltpu
from jax.experimental.pallas import tpu_sc as plsc

_N = 10000
_E = 160000
_D = 256
_H = 8
_C = _D // _H
_HALF = _D // 2

_EREAL = _E + _N
_K = 112
_NCH = 96
_T = _NCH * _K
_EP = 16 * _T
_PAD = _EP - _EREAL
_NT = 16
_NPB = 624
_NTAIL = _N - _NT * _NPB
_NB = 48

_ROW_BLK = 2000


def _mm_body(xlo_ref, xhi_ref, w_ref, att_ref, h2_ref, a_ref):
    h = jnp.dot(xlo_ref[...], w_ref[:_HALF, :],
                preferred_element_type=jnp.float32)
    h += jnp.dot(xhi_ref[...], w_ref[_HALF:, :],
                 preferred_element_type=jnp.float32)
    h2_ref[0] = h[:, :_HALF]
    h2_ref[1] = h[:, _HALF:]
    a_ref[...] = jnp.dot(h, att_ref[...], preferred_element_type=jnp.float32)


def _dense(xlo, xhi, W, att_cols):
    grid = (_N // _ROW_BLK,)
    return pl.pallas_call(
        _mm_body,
        grid=grid,
        in_specs=[
            pl.BlockSpec((_ROW_BLK, _HALF), lambda i: (i, 0)),
            pl.BlockSpec((_ROW_BLK, _HALF), lambda i: (i, 0)),
            pl.BlockSpec((_D, _D), lambda i: (0, 0)),
            pl.BlockSpec((_D, 2 * _H), lambda i: (0, 0)),
        ],
        out_specs=[
            pl.BlockSpec((2, _ROW_BLK, _HALF), lambda i: (0, i, 0)),
            pl.BlockSpec((_ROW_BLK, 2 * _H), lambda i: (i, 0)),
        ],
        out_shape=[
            jax.ShapeDtypeStruct((2, _N, _HALF), jnp.float32),
            jax.ShapeDtypeStruct((_N, 2 * _H), jnp.float32),
        ],
    )(xlo, xhi, W, att_cols)


def _sc_body(h2, a2d, srcp2, dstp, z16, z128, out,
             src_ch, dst_ch, exb0, exb1, abd0, abd1, rows0, rows1, denbuf,
             semi0, semi1, semg0, semg1, semx0, semx1, semr0, semr1,
             denom_sh, out_sh):
    c = lax.axis_index("c")
    s = lax.axis_index("s")
    lane = lax.iota(jnp.int32, 16)
    l4 = lane >> 2
    lh = lane & 3
    zero16 = lane * 0

    EXB = (exb0, exb1)
    ABD = (abd0, abd1)
    ROWS = (rows0, rows1)
    SEMI = (semi0, semi1)
    SEMG = (semg0, semg1)
    SEMX = (semx0, semx1)
    SEMR = (semr0, semr1)

    pltpu.sync_copy(z16.at[pl.ds(s * _NPB, _NPB)],
                    denom_sh.at[pl.ds(s * _NPB, _NPB)])
    pltpu.sync_copy(z128.at[pl.ds(s * _NPB, _NPB)],
                    out_sh.at[pl.ds(s * _NPB, _NPB)])

    @pl.when(s == 0)
    def _():
        pltpu.sync_copy(z16.at[pl.ds(_NT * _NPB, _NTAIL)],
                        denom_sh.at[pl.ds(_NT * _NPB, _NTAIL)])
        pltpu.sync_copy(z128.at[pl.ds(_NT * _NPB, _NTAIL)],
                        out_sh.at[pl.ds(_NT * _NPB, _NTAIL)])

    plsc.subcore_barrier()

    acol = 8 * c

    def issue_idx(ch, b):
        pltpu.async_copy(srcp2.at[c, s, ch], src_ch.at[b], SEMI[b])
        pltpu.async_copy(dstp.at[s, ch], dst_ch.at[b], SEMI[b])

    def wait_idx(b):
        pltpu.make_async_copy(srcp2.at[c, s, 0], src_ch.at[b], SEMI[b]).wait()
        pltpu.make_async_copy(dstp.at[s, 0], dst_ch.at[b], SEMI[b]).wait()

    def issue_dat(b):
        pltpu.async_copy(a2d.at[src_ch.at[b]], EXB[b], SEMG[b])
        pltpu.async_copy(a2d.at[dst_ch.at[b]], ABD[b], SEMG[b])
        pltpu.async_copy(h2.at[src_ch.at[b]], ROWS[b], SEMG[b])

    def wait_dat(b):
        pltpu.make_async_copy(a2d.at[src_ch.at[b]], EXB[b], SEMG[b]).wait()
        pltpu.make_async_copy(a2d.at[dst_ch.at[b]], ABD[b], SEMG[b]).wait()
        pltpu.make_async_copy(h2.at[src_ch.at[b]], ROWS[b], SEMG[b]).wait()

    def drain_scat(b):
        pltpu.make_async_copy(
            EXB[b], denom_sh.at[dst_ch.at[b]], SEMX[b]).wait()
        pltpu.make_async_copy(
            ROWS[b], out_sh.at[dst_ch.at[b]], SEMR[b]).wait()

    issue_idx(0, 0)
    wait_idx(0)
    issue_dat(0)
    issue_idx(1, 1)

    def step(ch, b):
        nb = 1 - b
        wait_dat(b)

        @pl.when(ch >= 1)
        def _():
            drain_scat(nb)

        @pl.when(ch + 1 < _NCH)
        def _():
            wait_idx(nb)
            issue_dat(nb)

        gbase = (s * _NCH + ch) * _K

        def quad(i, carry2):
            r = i * 4 + l4
            asv = plsc.load_gather(EXB[b], [r, acol + lh])
            adv = plsc.load_gather(ABD[b], [r, acol + 4 + lh])
            al = asv + adv
            al = jnp.where(al >= 0.0, al, al * 0.2)
            exv = jnp.exp(al)
            exv = jnp.where(gbase + r < _EREAL, exv, 0.0)
            plsc.store_scatter(EXB[b], [r, lh], exv)
            return carry2

        lax.fori_loop(0, _K // 4, quad, 0, unroll=2)
        pltpu.async_copy(EXB[b], denom_sh.at[dst_ch.at[b]], SEMX[b],
                         add=True)

        def edge(e, carry2):
            for hh in range(4):
                w = plsc.load_gather(EXB[b], [zero16 + e, zero16 + hh])
                for v in range(2):
                    sl = pl.ds((2 * hh + v) * 16, 16)
                    ROWS[b][e, sl] = ROWS[b][e, sl] * w
            return carry2

        lax.fori_loop(0, _K, edge, 0, unroll=2)
        pltpu.async_copy(ROWS[b], out_sh.at[dst_ch.at[b]], SEMR[b],
                         add=True)

        @pl.when(ch + 2 < _NCH)
        def _():
            issue_idx(ch + 2, b)

    def pair(i, carry):
        step(2 * i, 0)
        step(2 * i + 1, 1)
        return carry

    lax.fori_loop(0, _NCH // 2, pair, 0)
    drain_scat(1)
    plsc.subcore_barrier()

    def norm_block(base, nrows):
        pltpu.sync_copy(out_sh.at[pl.ds(base, nrows)],
                        rows0.at[pl.ds(0, nrows)])
        pltpu.sync_copy(denom_sh.at[pl.ds(base, nrows)],
                        denbuf.at[pl.ds(0, nrows)])

        def recip(i, carry2):
            r = (i * 16 + lane) >> 2
            cl = lane & 3
            dv = plsc.load_gather(denbuf, [r, cl])
            plsc.store_scatter(denbuf, [r, cl], 1.0 / dv)
            return carry2

        lax.fori_loop(0, nrows * 4 // 16, recip, 0)

        def row(rr, carry2):
            for hh in range(4):
                rv = plsc.load_gather(denbuf, [zero16 + rr, zero16 + hh])
                for v in range(2):
                    sl = pl.ds((2 * hh + v) * 16, 16)
                    rows0[rr, sl] = rows0[rr, sl] * rv
            return carry2

        lax.fori_loop(0, nrows, row, 0, unroll=2)
        pltpu.sync_copy(rows0.at[pl.ds(0, nrows)],
                        out.at[c, pl.ds(base, nrows)])

    def norm(bb, carry):
        norm_block(s * _NPB + bb * _NB, _NB)
        return carry

    lax.fori_loop(0, _NPB // _NB, norm, 0)

    @pl.when(s == 0)
    def _():
        norm_block(_NT * _NPB, _NTAIL)


_sc_call = pl.kernel(
    _sc_body,
    out_type=jax.ShapeDtypeStruct((2, _N, _HALF), jnp.float32),
    mesh=plsc.VectorSubcoreMesh(core_axis_name="c", subcore_axis_name="s"),
    compiler_params=pltpu.CompilerParams(
        use_tc_tiling_on_sc=False, needs_layout_passes=False),
    scratch_types=[
        pltpu.VMEM((2, _K), jnp.int32),
        pltpu.VMEM((2, _K), jnp.int32),
        pltpu.VMEM((_K, 2 * _H), jnp.float32),
        pltpu.VMEM((_K, 2 * _H), jnp.float32),
        pltpu.VMEM((_K, 2 * _H), jnp.float32),
        pltpu.VMEM((_K, 2 * _H), jnp.float32),
        pltpu.VMEM((_K, _HALF), jnp.float32),
        pltpu.VMEM((_K, _HALF), jnp.float32),
        pltpu.VMEM((_NB, 2 * _H), jnp.float32),
        pltpu.SemaphoreType.DMA,
        pltpu.SemaphoreType.DMA,
        pltpu.SemaphoreType.DMA,
        pltpu.SemaphoreType.DMA,
        pltpu.SemaphoreType.DMA,
        pltpu.SemaphoreType.DMA,
        pltpu.SemaphoreType.DMA,
        pltpu.SemaphoreType.DMA,
        pltpu.VMEM_SHARED((_N, 2 * _H), jnp.float32),
        pltpu.VMEM_SHARED((_N, _HALF), jnp.float32),
    ],
)


def kernel(x, edge_index, W, att_src, att_dst, bias):
    As = jnp.zeros((_H, _C, _H), jnp.float32).at[
        jnp.arange(_H), :, jnp.arange(_H)].set(att_src).reshape(_D, _H)
    Ad = jnp.zeros((_H, _C, _H), jnp.float32).at[
        jnp.arange(_H), :, jnp.arange(_H)].set(att_dst).reshape(_D, _H)
    att_cols = jnp.concatenate(
        [As[:, :4], Ad[:, :4], As[:, 4:], Ad[:, 4:]], axis=1)

    loops = jnp.arange(_N, dtype=jnp.int32)
    padi = jnp.arange(_PAD, dtype=jnp.int32)
    src = jnp.concatenate([edge_index[0], loops, (padi * 37) % _N])
    dst = jnp.concatenate([edge_index[1], loops, (padi * 41) % _N])
    srcp = src.reshape(_NT, _NCH, _K)
    srcp2 = jnp.stack([srcp, srcp + _N])
    dstp = dst.reshape(_NT, _NCH, _K)

    z16 = jnp.zeros((_N, 2 * _H), jnp.float32)
    z128 = jnp.zeros((_N, _HALF), jnp.float32)

    xlo = x[:, :_HALF]
    xhi = x[:, _HALF:]
    for _ in range(2):
        h2, a2 = _dense(xlo, xhi, W, att_cols)
        a2d = jnp.concatenate([a2, a2], axis=0)
        out2 = _sc_call(h2.reshape(2 * _N, _HALF), a2d, srcp2, dstp,
                        z16, z128)
        xlo = out2[0] + bias[:_HALF]
        xhi = out2[1] + bias[_HALF:]
    return jnp.concatenate([xlo, xhi], axis=1)


# speedup vs baseline: 50.4645x; 1.0061x over previous
import jax
import jax.numpy as jnp
from jax import lax
from jax.experimental import pallas as pl
from jax.experimental.pallas import tpu as pltpu
from jax.experimental.pallas import tpu_sc as plsc

_N = 10000
_E = 160000
_D = 256
_H = 8
_C = _D // _H
_HALF = _D // 2

_EREAL = _E + _N
_K = 112
_NCH = 96
_T = _NCH * _K
_EP = 16 * _T
_PAD = _EP - _EREAL
_NT = 16
_NPB = 624
_NTAIL = _N - _NT * _NPB
_NB = 48

_ROW_BLK = 2000


def _mm_body(xlo_ref, xhi_ref, w_ref, att_ref, h2_ref, a_ref):
    h = jnp.dot(xlo_ref[...], w_ref[:_HALF, :],
                preferred_element_type=jnp.float32)
    h += jnp.dot(xhi_ref[...], w_ref[_HALF:, :],
                 preferred_element_type=jnp.float32)
    h2_ref[0] = h[:, :_HALF]
    h2_ref[1] = h[:, _HALF:]
    a_ref[...] = jnp.dot(h, att_ref[...], preferred_element_type=jnp.float32)


def _dense(xlo, xhi, W, att_cols):
    grid = (_N // _ROW_BLK,)
    return pl.pallas_call(
        _mm_body,
        grid=grid,
        in_specs=[
            pl.BlockSpec((_ROW_BLK, _HALF), lambda i: (i, 0)),
            pl.BlockSpec((_ROW_BLK, _HALF), lambda i: (i, 0)),
            pl.BlockSpec((_D, _D), lambda i: (0, 0)),
            pl.BlockSpec((_D, 2 * _H), lambda i: (0, 0)),
        ],
        out_specs=[
            pl.BlockSpec((2, _ROW_BLK, _HALF), lambda i: (0, i, 0)),
            pl.BlockSpec((_ROW_BLK, 2 * _H), lambda i: (i, 0)),
        ],
        out_shape=[
            jax.ShapeDtypeStruct((2, _N, _HALF), jnp.float32),
            jax.ShapeDtypeStruct((_N, 2 * _H), jnp.float32),
        ],
    )(xlo, xhi, W, att_cols)


def _sc_body(h2, a2d, srcp2, dstp, z16, z128, out,
             src_ch, dst_ch, exb0, exb1, abd0, abd1, rows0, rows1, denbuf,
             semi0, semi1, semg0, semg1, semx0, semx1, semr0, semr1,
             denom_sh, out_sh):
    c = lax.axis_index("c")
    s = lax.axis_index("s")
    lane = lax.iota(jnp.int32, 16)
    l4 = lane >> 2
    lh = lane & 3
    zero16 = lane * 0

    EXB = (exb0, exb1)
    ABD = (abd0, abd1)
    ROWS = (rows0, rows1)
    SEMI = (semi0, semi1)
    SEMG = (semg0, semg1)
    SEMX = (semx0, semx1)
    SEMR = (semr0, semr1)

    pltpu.sync_copy(z16.at[pl.ds(s * _NPB, _NPB)],
                    denom_sh.at[pl.ds(s * _NPB, _NPB)])
    pltpu.sync_copy(z128.at[pl.ds(s * _NPB, _NPB)],
                    out_sh.at[pl.ds(s * _NPB, _NPB)])

    @pl.when(s == 0)
    def _():
        pltpu.sync_copy(z16.at[pl.ds(_NT * _NPB, _NTAIL)],
                        denom_sh.at[pl.ds(_NT * _NPB, _NTAIL)])
        pltpu.sync_copy(z128.at[pl.ds(_NT * _NPB, _NTAIL)],
                        out_sh.at[pl.ds(_NT * _NPB, _NTAIL)])

    plsc.subcore_barrier()

    acol = 8 * c

    def issue_idx(ch, b):
        pltpu.async_copy(srcp2.at[c, s, ch], src_ch.at[b], SEMI[b])
        pltpu.async_copy(dstp.at[s, ch], dst_ch.at[b], SEMI[b])

    def wait_idx(b):
        pltpu.make_async_copy(srcp2.at[c, s, 0], src_ch.at[b], SEMI[b]).wait()
        pltpu.make_async_copy(dstp.at[s, 0], dst_ch.at[b], SEMI[b]).wait()

    def issue_dat(b):
        pltpu.async_copy(a2d.at[src_ch.at[b]], EXB[b], SEMG[b])
        pltpu.async_copy(a2d.at[dst_ch.at[b]], ABD[b], SEMG[b])
        pltpu.async_copy(h2.at[src_ch.at[b]], ROWS[b], SEMG[b])

    def wait_dat(b):
        pltpu.make_async_copy(a2d.at[src_ch.at[b]], EXB[b], SEMG[b]).wait()
        pltpu.make_async_copy(a2d.at[dst_ch.at[b]], ABD[b], SEMG[b]).wait()
        pltpu.make_async_copy(h2.at[src_ch.at[b]], ROWS[b], SEMG[b]).wait()

    def drain_scat(b):
        pltpu.make_async_copy(
            EXB[b], denom_sh.at[dst_ch.at[b]], SEMX[b]).wait()
        pltpu.make_async_copy(
            ROWS[b], out_sh.at[dst_ch.at[b]], SEMR[b]).wait()

    issue_idx(0, 0)
    wait_idx(0)
    issue_dat(0)
    issue_idx(1, 1)

    def step(ch, b):
        nb = 1 - b
        wait_dat(b)

        @pl.when(ch >= 1)
        def _():
            drain_scat(nb)

        @pl.when(ch + 1 < _NCH)
        def _():
            wait_idx(nb)
            issue_dat(nb)

        gbase = (s * _NCH + ch) * _K

        def quad(i, carry2):
            r = i * 4 + l4
            asv = plsc.load_gather(EXB[b], [r, acol + lh])
            adv = plsc.load_gather(ABD[b], [r, acol + 4 + lh])
            al = asv + adv
            al = jnp.where(al >= 0.0, al, al * 0.2)
            exv = jnp.exp(al)
            exv = jnp.where(gbase + r < _EREAL, exv, 0.0)
            plsc.store_scatter(EXB[b], [r, lh], exv)
            return carry2

        lax.fori_loop(0, _K // 4, quad, 0, unroll=4)
        pltpu.async_copy(EXB[b], denom_sh.at[dst_ch.at[b]], SEMX[b],
                         add=True)

        def edge(e, carry2):
            for hh in range(4):
                w = plsc.load_gather(EXB[b], [zero16 + e, zero16 + hh])
                for v in range(2):
                    sl = pl.ds((2 * hh + v) * 16, 16)
                    ROWS[b][e, sl] = ROWS[b][e, sl] * w
            return carry2

        lax.fori_loop(0, _K, edge, 0, unroll=4)
        pltpu.async_copy(ROWS[b], out_sh.at[dst_ch.at[b]], SEMR[b],
                         add=True)

        @pl.when(ch + 2 < _NCH)
        def _():
            issue_idx(ch + 2, b)

    def pair(i, carry):
        step(2 * i, 0)
        step(2 * i + 1, 1)
        return carry

    lax.fori_loop(0, _NCH // 2, pair, 0)
    drain_scat(1)
    plsc.subcore_barrier()

    def norm_block(base, nrows):
        pltpu.sync_copy(out_sh.at[pl.ds(base, nrows)],
                        rows0.at[pl.ds(0, nrows)])
        pltpu.sync_copy(denom_sh.at[pl.ds(base, nrows)],
                        denbuf.at[pl.ds(0, nrows)])

        def recip(i, carry2):
            r = (i * 16 + lane) >> 2
            cl = lane & 3
            dv = plsc.load_gather(denbuf, [r, cl])
            plsc.store_scatter(denbuf, [r, cl], 1.0 / dv)
            return carry2

        lax.fori_loop(0, nrows * 4 // 16, recip, 0)

        def row(rr, carry2):
            for hh in range(4):
                rv = plsc.load_gather(denbuf, [zero16 + rr, zero16 + hh])
                for v in range(2):
                    sl = pl.ds((2 * hh + v) * 16, 16)
                    rows0[rr, sl] = rows0[rr, sl] * rv
            return carry2

        lax.fori_loop(0, nrows, row, 0, unroll=2)
        pltpu.sync_copy(rows0.at[pl.ds(0, nrows)],
                        out.at[c, pl.ds(base, nrows)])

    def norm(bb, carry):
        norm_block(s * _NPB + bb * _NB, _NB)
        return carry

    lax.fori_loop(0, _NPB // _NB, norm, 0)

    @pl.when(s == 0)
    def _():
        norm_block(_NT * _NPB, _NTAIL)


_sc_call = pl.kernel(
    _sc_body,
    out_type=jax.ShapeDtypeStruct((2, _N, _HALF), jnp.float32),
    mesh=plsc.VectorSubcoreMesh(core_axis_name="c", subcore_axis_name="s"),
    compiler_params=pltpu.CompilerParams(
        use_tc_tiling_on_sc=False, needs_layout_passes=False),
    scratch_types=[
        pltpu.VMEM((2, _K), jnp.int32),
        pltpu.VMEM((2, _K), jnp.int32),
        pltpu.VMEM((_K, 2 * _H), jnp.float32),
        pltpu.VMEM((_K, 2 * _H), jnp.float32),
        pltpu.VMEM((_K, 2 * _H), jnp.float32),
        pltpu.VMEM((_K, 2 * _H), jnp.float32),
        pltpu.VMEM((_K, _HALF), jnp.float32),
        pltpu.VMEM((_K, _HALF), jnp.float32),
        pltpu.VMEM((_NB, 2 * _H), jnp.float32),
        pltpu.SemaphoreType.DMA,
        pltpu.SemaphoreType.DMA,
        pltpu.SemaphoreType.DMA,
        pltpu.SemaphoreType.DMA,
        pltpu.SemaphoreType.DMA,
        pltpu.SemaphoreType.DMA,
        pltpu.SemaphoreType.DMA,
        pltpu.SemaphoreType.DMA,
        pltpu.VMEM_SHARED((_N, 2 * _H), jnp.float32),
        pltpu.VMEM_SHARED((_N, _HALF), jnp.float32),
    ],
)


def kernel(x, edge_index, W, att_src, att_dst, bias):
    As = jnp.zeros((_H, _C, _H), jnp.float32).at[
        jnp.arange(_H), :, jnp.arange(_H)].set(att_src).reshape(_D, _H)
    Ad = jnp.zeros((_H, _C, _H), jnp.float32).at[
        jnp.arange(_H), :, jnp.arange(_H)].set(att_dst).reshape(_D, _H)
    att_cols = jnp.concatenate(
        [As[:, :4], Ad[:, :4], As[:, 4:], Ad[:, 4:]], axis=1)

    loops = jnp.arange(_N, dtype=jnp.int32)
    padi = jnp.arange(_PAD, dtype=jnp.int32)
    src = jnp.concatenate([edge_index[0], loops, (padi * 37) % _N])
    dst = jnp.concatenate([edge_index[1], loops, (padi * 41) % _N])
    srcp = src.reshape(_NT, _NCH, _K)
    srcp2 = jnp.stack([srcp, srcp + _N])
    dstp = dst.reshape(_NT, _NCH, _K)

    z16 = jnp.zeros((_N, 2 * _H), jnp.float32)
    z128 = jnp.zeros((_N, _HALF), jnp.float32)

    xlo = x[:, :_HALF]
    xhi = x[:, _HALF:]
    for _ in range(2):
        h2, a2 = _dense(xlo, xhi, W, att_cols)
        a2d = jnp.concatenate([a2, a2], axis=0)
        out2 = _sc_call(h2.reshape(2 * _N, _HALF), a2d, srcp2, dstp,
                        z16, z128)
        xlo = out2[0] + bias[:_HALF]
        xhi = out2[1] + bias[_HALF:]
    return jnp.concatenate([xlo, xhi], axis=1)
